# baseline (device time: 308674 ns/iter reference)
import jax
import jax.numpy as jnp
from jax import lax
from jax.experimental import pallas as pl
from jax.experimental.pallas import tpu as pltpu

N_DEV = 16
SQ = 512
D = 1024
NH = 8
DH = 128
SCALE = 0.08838834764831843


def _contrib(x_t, wqkv, wo):
    qkv = jnp.dot(x_t, wqkv, preferred_element_type=jnp.float32).astype(
        jnp.bfloat16)
    os = []
    for h in range(NH):
        qh = qkv[:, h * DH:(h + 1) * DH]
        kh = qkv[:, D + h * DH:D + (h + 1) * DH]
        vh = qkv[:, 2 * D + h * DH:2 * D + (h + 1) * DH]
        s = lax.dot_general(
            qh, kh, (((1,), (1,)), ((), ())),
            preferred_element_type=jnp.float32,
        ) * SCALE
        p = jnp.exp(s)
        l = jnp.sum(p, axis=-1, keepdims=True)
        o = jnp.dot(p.astype(jnp.bfloat16), vh, preferred_element_type=jnp.float32)
        os.append((o / l).astype(jnp.bfloat16))
    return jnp.dot(jnp.concatenate(os, axis=1), wo,
                   preferred_element_type=jnp.float32)


def kernel(x, Wq, Wo, Wk, Wv):
    xb = x.reshape(SQ, D).astype(jnp.bfloat16)
    wqkvb = jnp.concatenate(
        [Wq.astype(jnp.bfloat16), Wk.astype(jnp.bfloat16),
         Wv.astype(jnp.bfloat16)], axis=1)
    wob = Wo.astype(jnp.bfloat16)

    def body(x_ref, wqkv_ref, wo_ref, out_ref,
             xgat, psend, precv, xs_sem, xr_sem, xsl_sem, xrl_sem,
             ps_sem, pr_sem):
        my = lax.axis_index("i")
        left = lax.rem(my + N_DEV - 1, N_DEV)
        right = lax.rem(my + 1, N_DEV)

        def x_r_hop(r):
            return pltpu.make_async_remote_copy(
                src_ref=xgat.at[r],
                dst_ref=xgat.at[r + 1],
                send_sem=xs_sem.at[r],
                recv_sem=xr_sem.at[r],
                device_id=(right,),
                device_id_type=pl.DeviceIdType.MESH,
            )

        def x_l_hop(l):
            return pltpu.make_async_remote_copy(
                src_ref=xgat.at[lax.rem(N_DEV - l, N_DEV)],
                dst_ref=xgat.at[N_DEV - 1 - l],
                send_sem=xsl_sem.at[l],
                recv_sem=xrl_sem.at[l],
                device_id=(left,),
                device_id_type=pl.DeviceIdType.MESH,
            )

        def p_hop(s):
            return pltpu.make_async_remote_copy(
                src_ref=psend.at[lax.rem(s, 4)],
                dst_ref=precv.at[s + 1],
                send_sem=ps_sem.at[s],
                recv_sem=pr_sem.at[s],
                device_id=(right,),
                device_id_type=pl.DeviceIdType.MESH,
            )

        barrier = pltpu.get_barrier_semaphore()
        pl.semaphore_signal(barrier, inc=1, device_id=(left,),
                            device_id_type=pl.DeviceIdType.MESH)
        pl.semaphore_signal(barrier, inc=1, device_id=(right,),
                            device_id_type=pl.DeviceIdType.MESH)
        pl.semaphore_wait(barrier, 2)

        xgat[0] = x_ref[...]
        precv[0] = jnp.zeros((SQ, D), jnp.bfloat16)
        x_r_hop(0).start()
        x_l_hop(0).start()

        def step(s, carry):
            @pl.when(s < 8)
            def _():
                x_l_hop(lax.min(s, 7)).wait_recv()

            @pl.when(s < 7)
            def _():
                x_l_hop(lax.min(s + 1, 7)).start()
                x_r_hop(lax.min(s, 6)).wait_recv()

            @pl.when(s < 6)
            def _():
                x_r_hop(lax.min(s + 1, 6)).start()

            c = _contrib(xgat[s + 1], wqkv_ref[...], wo_ref[...])

            @pl.when(s > 0)
            def _():
                p_hop(s - 1).wait_recv()

            c = c + precv[s].astype(jnp.float32)

            @pl.when(s >= 4)
            def _():
                p_hop(s - 4).wait_send()

            psend[lax.rem(s, 4)] = c.astype(jnp.bfloat16)
            p_hop(s).start()
            return carry

        lax.fori_loop(0, N_DEV - 1, step, 0)

        own = _contrib(x_ref[...], wqkv_ref[...], wo_ref[...])
        p_hop(N_DEV - 2).wait_recv()
        out_ref[...] = own + precv[N_DEV - 1].astype(jnp.float32)

        def drain_xr(s, carry):
            x_r_hop(s).wait_send()
            return carry

        lax.fori_loop(0, 7, drain_xr, 0)

        def drain_xl(s, carry):
            x_l_hop(s).wait_send()
            return carry

        lax.fori_loop(0, 8, drain_xl, 0)

        def drain_p(s, carry):
            p_hop(s).wait_send()
            return carry

        lax.fori_loop(N_DEV - 5, N_DEV - 1, drain_p, 0)

    out = pl.pallas_call(
        body,
        out_shape=jax.ShapeDtypeStruct((SQ, D), jnp.float32),
        in_specs=[pl.BlockSpec(memory_space=pltpu.VMEM)] * 3,
        out_specs=pl.BlockSpec(memory_space=pltpu.VMEM),
        scratch_shapes=[
            pltpu.VMEM((N_DEV, SQ, D), jnp.bfloat16),
            pltpu.VMEM((4, SQ, D), jnp.bfloat16),
            pltpu.VMEM((N_DEV, SQ, D), jnp.bfloat16),
            pltpu.SemaphoreType.DMA((7,)),
            pltpu.SemaphoreType.DMA((7,)),
            pltpu.SemaphoreType.DMA((8,)),
            pltpu.SemaphoreType.DMA((8,)),
            pltpu.SemaphoreType.DMA((N_DEV - 1,)),
            pltpu.SemaphoreType.DMA((N_DEV - 1,)),
        ],
        compiler_params=pltpu.CompilerParams(
            collective_id=0,
            vmem_limit_bytes=100 * 1024 * 1024,
        ),
    )(xb, wqkvb, wob)
    return out.reshape(1, SQ, D)


# device time: 297508 ns/iter; 1.0375x vs baseline; 1.0375x over previous
import jax
import jax.numpy as jnp
from jax import lax
from jax.experimental import pallas as pl
from jax.experimental.pallas import tpu as pltpu

N_DEV = 16
SQ = 512
D = 1024
NH = 8
DH = 128
SCALE = 0.08838834764831843


def _contrib(x_t, wqkv, wo):
    qkv = jnp.dot(x_t, wqkv, preferred_element_type=jnp.float32).astype(
        jnp.bfloat16)
    acc = None
    for h in range(NH):
        qh = qkv[:, h * DH:(h + 1) * DH]
        kh = qkv[:, D + h * DH:D + (h + 1) * DH]
        vh = qkv[:, 2 * D + h * DH:2 * D + (h + 1) * DH]
        s = lax.dot_general(
            qh, kh, (((1,), (1,)), ((), ())),
            preferred_element_type=jnp.float32,
        ) * SCALE
        p = jnp.exp(s)
        l = jnp.sum(p, axis=-1, keepdims=True)
        o = jnp.dot(p.astype(jnp.bfloat16), vh, preferred_element_type=jnp.float32)
        o = (o / l).astype(jnp.bfloat16)
        c = jnp.dot(o, wo[h * DH:(h + 1) * DH, :],
                    preferred_element_type=jnp.float32)
        acc = c if acc is None else acc + c
    return acc


def kernel(x, Wq, Wo, Wk, Wv):
    xb = x.reshape(SQ, D).astype(jnp.bfloat16)
    wqkvb = jnp.concatenate(
        [Wq.astype(jnp.bfloat16), Wk.astype(jnp.bfloat16),
         Wv.astype(jnp.bfloat16)], axis=1)
    wob = Wo.astype(jnp.bfloat16)

    def body(x_ref, wqkv_ref, wo_ref, out_ref,
             xgat, psend, precv, xs_sem, xr_sem, xsl_sem, xrl_sem,
             ps_sem, pr_sem):
        my = lax.axis_index("i")
        left = lax.rem(my + N_DEV - 1, N_DEV)
        right = lax.rem(my + 1, N_DEV)

        def x_r_hop(r):
            return pltpu.make_async_remote_copy(
                src_ref=xgat.at[r],
                dst_ref=xgat.at[r + 1],
                send_sem=xs_sem.at[r],
                recv_sem=xr_sem.at[r],
                device_id=(right,),
                device_id_type=pl.DeviceIdType.MESH,
            )

        def x_l_hop(l):
            return pltpu.make_async_remote_copy(
                src_ref=xgat.at[lax.rem(N_DEV - l, N_DEV)],
                dst_ref=xgat.at[N_DEV - 1 - l],
                send_sem=xsl_sem.at[l],
                recv_sem=xrl_sem.at[l],
                device_id=(left,),
                device_id_type=pl.DeviceIdType.MESH,
            )

        def p_hop(s):
            return pltpu.make_async_remote_copy(
                src_ref=psend.at[lax.rem(s, 4)],
                dst_ref=precv.at[s + 1],
                send_sem=ps_sem.at[s],
                recv_sem=pr_sem.at[s],
                device_id=(right,),
                device_id_type=pl.DeviceIdType.MESH,
            )

        barrier = pltpu.get_barrier_semaphore()
        pl.semaphore_signal(barrier, inc=1, device_id=(left,),
                            device_id_type=pl.DeviceIdType.MESH)
        pl.semaphore_signal(barrier, inc=1, device_id=(right,),
                            device_id_type=pl.DeviceIdType.MESH)
        pl.semaphore_wait(barrier, 2)

        xgat[0] = x_ref[...]
        precv[0] = jnp.zeros((SQ, D), jnp.bfloat16)
        x_r_hop(0).start()
        x_l_hop(0).start()

        def step(s, carry):
            @pl.when(s < 8)
            def _():
                x_l_hop(lax.min(s, 7)).wait_recv()

            @pl.when(s < 7)
            def _():
                x_l_hop(lax.min(s + 1, 7)).start()
                x_r_hop(lax.min(s, 6)).wait_recv()

            @pl.when(s < 6)
            def _():
                x_r_hop(lax.min(s + 1, 6)).start()

            c = _contrib(xgat[s + 1], wqkv_ref[...], wo_ref[...])

            @pl.when(s > 0)
            def _():
                p_hop(s - 1).wait_recv()

            c = c + precv[s].astype(jnp.float32)

            @pl.when(s >= 4)
            def _():
                p_hop(s - 4).wait_send()

            psend[lax.rem(s, 4)] = c.astype(jnp.bfloat16)
            p_hop(s).start()
            return carry

        lax.fori_loop(0, N_DEV - 1, step, 0)

        own = _contrib(x_ref[...], wqkv_ref[...], wo_ref[...])
        p_hop(N_DEV - 2).wait_recv()
        out_ref[...] = own + precv[N_DEV - 1].astype(jnp.float32)

        def drain_xr(s, carry):
            x_r_hop(s).wait_send()
            return carry

        lax.fori_loop(0, 7, drain_xr, 0)

        def drain_xl(s, carry):
            x_l_hop(s).wait_send()
            return carry

        lax.fori_loop(0, 8, drain_xl, 0)

        def drain_p(s, carry):
            p_hop(s).wait_send()
            return carry

        lax.fori_loop(N_DEV - 5, N_DEV - 1, drain_p, 0)

    out = pl.pallas_call(
        body,
        out_shape=jax.ShapeDtypeStruct((SQ, D), jnp.float32),
        in_specs=[pl.BlockSpec(memory_space=pltpu.VMEM)] * 3,
        out_specs=pl.BlockSpec(memory_space=pltpu.VMEM),
        scratch_shapes=[
            pltpu.VMEM((N_DEV, SQ, D), jnp.bfloat16),
            pltpu.VMEM((4, SQ, D), jnp.bfloat16),
            pltpu.VMEM((N_DEV, SQ, D), jnp.bfloat16),
            pltpu.SemaphoreType.DMA((7,)),
            pltpu.SemaphoreType.DMA((7,)),
            pltpu.SemaphoreType.DMA((8,)),
            pltpu.SemaphoreType.DMA((8,)),
            pltpu.SemaphoreType.DMA((N_DEV - 1,)),
            pltpu.SemaphoreType.DMA((N_DEV - 1,)),
        ],
        compiler_params=pltpu.CompilerParams(
            collective_id=0,
            vmem_limit_bytes=100 * 1024 * 1024,
        ),
    )(xb, wqkvb, wob)
    return out.reshape(1, SQ, D)
